# Initial kernel scaffold; baseline (speedup 1.0000x reference)
import jax
import jax.numpy as jnp
from jax.experimental import pallas as pl


def kernel(inputs, P):
    raise NotImplementedError("write your pallas kernel here")


# speedup vs baseline: 1.7276x; 1.7276x over previous
import jax
import jax.numpy as jnp
from jax.experimental import pallas as pl

_BS = 512


def _pe_add_body(x_ref, p_ref, o_ref):
    p = p_ref[...]
    for b in range(x_ref.shape[0]):
        o_ref[b] = x_ref[b] + p


def kernel(inputs, P):
    B, S, D = inputs.shape
    p_used = P[:S]
    grid = (S // _BS,)
    return pl.pallas_call(
        _pe_add_body,
        grid=grid,
        in_specs=[
            pl.BlockSpec((B, _BS, D), lambda i: (0, i, 0)),
            pl.BlockSpec((_BS, D), lambda i: (i, 0)),
        ],
        out_specs=pl.BlockSpec((B, _BS, D), lambda i: (0, i, 0)),
        out_shape=jax.ShapeDtypeStruct((B, S, D), inputs.dtype),
    )(inputs, p_used)
